# baseline (speedup 1.0000x reference)
import functools

import jax
import jax.numpy as jnp
from jax import lax
from jax.experimental import pallas as pl
from jax.experimental.pallas import tpu as pltpu
from jax.experimental.pallas import tpu_sc as plsc

K = 10

NC = 2
NS = 16
L = 16
NW = NC * NS


def _knn_kernel(n_total, rows_per_tile, xc_ref, x_ref, idx_ref, xx_ref):
    rt = pl.program_id(0)
    xall = x_ref[...]

    @pl.when(rt == 0)
    def _():
        xx_ref[0, :] = jnp.sum(xall * xall, axis=0)

    s = 2.0 * lax.dot_general(
        xc_ref[...], xall, (((0,), (0,)), ((), ())),
        preferred_element_type=jnp.float32)
    s = s - xx_ref[0, :][None, :]
    iota = lax.broadcasted_iota(jnp.int32, s.shape, 1)
    neg = jnp.float32(jnp.finfo(jnp.float32).min)
    self_col = rt * rows_per_tile + lax.broadcasted_iota(
        jnp.int32, (rows_per_tile, 1), 0)
    cols = [self_col]
    s = jnp.where(iota == self_col, neg, s)
    for _ in range(K - 1):
        m = jnp.max(s, axis=1, keepdims=True)
        cand = jnp.where(s == m, iota, n_total)
        a = jnp.min(cand, axis=1)
        cols.append(a[:, None])
        s = jnp.where(iota == a[:, None], neg, s)
    idx_ref[...] = jnp.concatenate(cols, axis=1)


def _uw_kernel(x_ref, w1t_ref, w2t_ref, u_ref, w_ref):
    xb = x_ref[0]
    dn = (((0,), (0,)), ((), ()))
    u = lax.dot_general(xb, w1t_ref[...], dn,
                        preferred_element_type=jnp.float32)
    v = lax.dot_general(xb, w2t_ref[...], dn,
                        preferred_element_type=jnp.float32)
    u_ref[0] = u
    w_ref[0] = v - u


def _final_kernel(n_total, tile, n_stats, gmax_ref, gmin_ref, w_ref,
                  stats_ref, gam_ref, bet_ref, out_ref):
    st = jnp.sum(stats_ref[...], axis=0)
    s1 = st[0:1, :]
    s2 = st[1:2, :]
    cr = st[2:3, :]
    sw = st[3:4, :]
    sw2 = st[4:5, :]
    inv_m = jnp.float32(1.0 / n_stats)
    mean = (s1 + K * sw) * inv_m
    ey2 = (s2 + 2.0 * cr + K * sw2) * inv_m
    var = ey2 - mean * mean
    inv = lax.rsqrt(var + 1e-5)
    a = gam_ref[...] * inv
    bsh = bet_ref[...] - a * mean
    cond = jnp.broadcast_to(a >= 0.0, gmax_ref.shape)
    z = jnp.where(cond, gmax_ref[...], gmin_ref[...]) + w_ref[...]
    y = a * z + bsh
    y = jnp.where(y >= 0.0, y, 0.2 * y)
    out_ref[0] = jnp.transpose(y, (1, 0))


def _make_sc_gather(bn, o):
    pts = bn // NW
    p_sub = 8
    idxc = p_sub * K
    n_sub = pts // p_sub
    nch = o // L
    mesh = plsc.VectorSubcoreMesh(core_axis_name="c", subcore_axis_name="s")

    @functools.partial(
        pl.kernel,
        mesh=mesh,
        out_type=[
            jax.ShapeDtypeStruct((bn, o), jnp.float32),
            jax.ShapeDtypeStruct((bn, o), jnp.float32),
            jax.ShapeDtypeStruct((NW, 8, o), jnp.float32),
        ],
        scratch_types=[
            pltpu.VMEM((pts * K,), jnp.int32),
            pltpu.VMEM((idxc, o), jnp.float32),
            pltpu.VMEM((idxc, o), jnp.float32),
            pltpu.VMEM((p_sub, o), jnp.float32),
            pltpu.VMEM((p_sub, o), jnp.float32),
            pltpu.VMEM((p_sub, o), jnp.float32),
            pltpu.VMEM((8, o), jnp.float32),
            pltpu.SemaphoreType.DMA,
            pltpu.SemaphoreType.DMA,
        ],
    )
    def gather_kernel(u_hbm, w_hbm, idx_hbm, gmax_hbm, gmin_hbm, stats_hbm,
                      idx_v, rows_a, rows_b, w_v, gmax_v, gmin_v, stats_v,
                      sem_a, sem_b):
        wid = lax.axis_index("s") * NC + lax.axis_index("c")
        zeros = jnp.zeros((L,), jnp.float32)
        for r in range(8):
            for c in range(nch):
                stats_v[r, pl.ds(c * L, L)] = zeros

        pltpu.sync_copy(idx_hbm.at[pl.ds(wid * pts * K, pts * K)], idx_v)
        bufs = (rows_a, rows_b)
        sems = (sem_a, sem_b)
        pltpu.async_copy(u_hbm.at[idx_v.at[pl.ds(0, idxc)]], rows_a, sem_a)

        def process(g, rows_v):
            pbase = wid * pts + g * p_sub
            pltpu.sync_copy(w_hbm.at[pl.ds(pbase, p_sub)], w_v)
            for c in range(nch):
                sl = pl.ds(c * L, L)

                def point(p, st):
                    s1, s2, cr, sw, sw2 = st
                    v0 = rows_v[p * K + 0, sl]
                    mx = v0
                    mn = v0
                    sm = v0
                    sq = v0 * v0
                    for j in range(1, K):
                        vj = rows_v[p * K + j, sl]
                        mx = jnp.maximum(mx, vj)
                        mn = jnp.minimum(mn, vj)
                        sm = sm + vj
                        sq = sq + vj * vj
                    gmax_v[p, sl] = mx
                    gmin_v[p, sl] = mn
                    wp = w_v[p, sl]
                    return (s1 + sm, s2 + sq, cr + wp * sm, sw + wp,
                            sw2 + wp * wp)

                st = lax.fori_loop(
                    0, p_sub, point, (zeros, zeros, zeros, zeros, zeros),
                    unroll=4)
                stats_v[0, sl] = stats_v[0, sl] + st[0]
                stats_v[1, sl] = stats_v[1, sl] + st[1]
                stats_v[2, sl] = stats_v[2, sl] + st[2]
                stats_v[3, sl] = stats_v[3, sl] + st[3]
                stats_v[4, sl] = stats_v[4, sl] + st[4]
            pltpu.sync_copy(gmax_v, gmax_hbm.at[pl.ds(pbase, p_sub)])
            pltpu.sync_copy(gmin_v, gmin_hbm.at[pl.ds(pbase, p_sub)])

        def pair(q, carry):
            for hb in range(2):
                g = q * 2 + hb
                pltpu.make_async_copy(
                    u_hbm.at[idx_v.at[pl.ds(0, idxc)]], bufs[hb],
                    sems[hb]).wait()

                @pl.when(g + 1 < n_sub)
                def _():
                    pltpu.async_copy(
                        u_hbm.at[idx_v.at[pl.ds((g + 1) * idxc, idxc)]],
                        bufs[1 - hb], sems[1 - hb])

                process(g, bufs[hb])
            return carry

        lax.fori_loop(0, n_sub // 2, pair, 0)
        pltpu.sync_copy(stats_v, stats_hbm.at[wid])

    return gather_kernel


def kernel(x, W, gamma, beta):
    B, C, N = x.shape
    O = W.shape[0]
    BN = B * N
    w1t = jnp.transpose(W[:, :C])
    w2t = jnp.transpose(W[:, C:])

    u, w = pl.pallas_call(
        _uw_kernel,
        grid=(B,),
        in_specs=[
            pl.BlockSpec((1, C, N), lambda b: (b, 0, 0)),
            pl.BlockSpec((C, O), lambda b: (0, 0)),
            pl.BlockSpec((C, O), lambda b: (0, 0)),
        ],
        out_specs=[
            pl.BlockSpec((1, N, O), lambda b: (b, 0, 0)),
            pl.BlockSpec((1, N, O), lambda b: (b, 0, 0)),
        ],
        out_shape=[
            jax.ShapeDtypeStruct((B, N, O), jnp.float32),
            jax.ShapeDtypeStruct((B, N, O), jnp.float32),
        ],
    )(x, w1t, w2t)

    RT = 1024
    knn_call = pl.pallas_call(
        functools.partial(_knn_kernel, N, RT),
        grid=(N // RT,),
        in_specs=[
            pl.BlockSpec((C, RT), lambda r: (0, r)),
            pl.BlockSpec((C, N), lambda r: (0, 0)),
        ],
        out_specs=pl.BlockSpec((RT, K), lambda r: (r, 0)),
        out_shape=jax.ShapeDtypeStruct((N, K), jnp.int32),
        scratch_shapes=[pltpu.VMEM((1, N), jnp.float32)],
    )
    gather_call = _make_sc_gather(N, O)

    gmax_l, gmin_l, stats_l = [], [], []
    for b in range(B):
        idx_b = knn_call(x[b], x[b])
        gx, gn, st = gather_call(u[b], w[b], idx_b.reshape(N * K))
        gmax_l.append(gx)
        gmin_l.append(gn)
        stats_l.append(st)

    gmax = jnp.concatenate(gmax_l, axis=0)
    gmin = jnp.concatenate(gmin_l, axis=0)
    stats = jnp.concatenate(stats_l, axis=0)
    wf = w.reshape(BN, O)

    FT = 512
    out = pl.pallas_call(
        functools.partial(_final_kernel, N, FT, BN * K),
        grid=(BN // FT,),
        in_specs=[
            pl.BlockSpec((FT, O), lambda i: (i, 0)),
            pl.BlockSpec((FT, O), lambda i: (i, 0)),
            pl.BlockSpec((FT, O), lambda i: (i, 0)),
            pl.BlockSpec((B * NW, 8, O), lambda i: (0, 0, 0)),
            pl.BlockSpec((1, O), lambda i: (0, 0)),
            pl.BlockSpec((1, O), lambda i: (0, 0)),
        ],
        out_specs=pl.BlockSpec(
            (1, O, FT), lambda i: (i // (N // FT), 0, i % (N // FT))
        ),
        out_shape=jax.ShapeDtypeStruct((B, O, N), jnp.float32),
    )(gmax, gmin, wf, stats, gamma.reshape(1, O), beta.reshape(1, O))

    return out


# speedup vs baseline: 1.0886x; 1.0886x over previous
import functools

import jax
import jax.numpy as jnp
from jax import lax
from jax.experimental import pallas as pl
from jax.experimental.pallas import tpu as pltpu
from jax.experimental.pallas import tpu_sc as plsc

K = 10

NC = 2
NS = 16
L = 16
NW = NC * NS


def _knn_kernel(n_total, rows_per_tile, xc_ref, x_ref, idx_ref, xx_ref):
    rt = pl.program_id(0)
    xall = x_ref[...]

    @pl.when(rt == 0)
    def _():
        xx_ref[0, :] = jnp.sum(xall * xall, axis=0)

    s = 2.0 * lax.dot_general(
        xc_ref[...], xall, (((0,), (0,)), ((), ())),
        preferred_element_type=jnp.float32)
    s = s - xx_ref[0, :][None, :]
    iota = lax.broadcasted_iota(jnp.int32, s.shape, 1)
    neg = jnp.float32(jnp.finfo(jnp.float32).min)
    self_col = rt * rows_per_tile + lax.broadcasted_iota(
        jnp.int32, (rows_per_tile, 1), 0)
    cols = [self_col]
    s = jnp.where(iota == self_col, neg, s)
    for _ in range(K - 1):
        m = jnp.max(s, axis=1, keepdims=True)
        cand = jnp.where(s == m, iota, n_total)
        a = jnp.min(cand, axis=1)
        cols.append(a[:, None])
        s = jnp.where(iota == a[:, None], neg, s)
    idx_ref[...] = jnp.concatenate(cols, axis=1)


def _uw_kernel(x_ref, w1t_ref, w2t_ref, u_ref, w_ref):
    xb = x_ref[0]
    dn = (((0,), (0,)), ((), ()))
    u = lax.dot_general(xb, w1t_ref[...], dn,
                        preferred_element_type=jnp.float32)
    v = lax.dot_general(xb, w2t_ref[...], dn,
                        preferred_element_type=jnp.float32)
    u_ref[0] = u
    w_ref[0] = v - u


def _final_kernel(n_total, tile, n_stats, gmax_ref, gmin_ref, w_ref,
                  stats_ref, gam_ref, bet_ref, out_ref):
    st = jnp.sum(stats_ref[...], axis=0)
    s1 = st[0:1, :]
    s2 = st[1:2, :]
    cr = st[2:3, :]
    sw = st[3:4, :]
    sw2 = st[4:5, :]
    inv_m = jnp.float32(1.0 / n_stats)
    mean = (s1 + K * sw) * inv_m
    ey2 = (s2 + 2.0 * cr + K * sw2) * inv_m
    var = ey2 - mean * mean
    inv = lax.rsqrt(var + 1e-5)
    a = gam_ref[...] * inv
    bsh = bet_ref[...] - a * mean
    cond = jnp.broadcast_to(a >= 0.0, gmax_ref.shape)
    z = jnp.where(cond, gmax_ref[...], gmin_ref[...]) + w_ref[...]
    y = a * z + bsh
    y = jnp.where(y >= 0.0, y, 0.2 * y)
    out_ref[0] = jnp.transpose(y, (1, 0))


def _make_sc_gather(bn, o):
    pts = bn // NW
    p_sub = 8
    idxc = p_sub * K
    n_sub = pts // p_sub
    nch = o // L
    mesh = plsc.VectorSubcoreMesh(core_axis_name="c", subcore_axis_name="s")

    @functools.partial(
        pl.kernel,
        mesh=mesh,
        out_type=[
            jax.ShapeDtypeStruct((bn, o), jnp.float32),
            jax.ShapeDtypeStruct((bn, o), jnp.float32),
            jax.ShapeDtypeStruct((NW, 8, o), jnp.float32),
        ],
        scratch_types=[
            pltpu.VMEM((pts * K,), jnp.int32),
            pltpu.VMEM((idxc, o), jnp.float32),
            pltpu.VMEM((idxc, o), jnp.float32),
            pltpu.VMEM((p_sub, o), jnp.float32),
            pltpu.VMEM((p_sub, o), jnp.float32),
            pltpu.VMEM((p_sub, o), jnp.float32),
            pltpu.VMEM((8, o), jnp.float32),
            pltpu.SemaphoreType.DMA,
            pltpu.SemaphoreType.DMA,
        ],
    )
    def gather_kernel(u_hbm, w_hbm, idx_hbm, gmax_hbm, gmin_hbm, stats_hbm,
                      idx_v, rows_a, rows_b, w_v, gmax_v, gmin_v, stats_v,
                      sem_a, sem_b):
        wid = lax.axis_index("s") * NC + lax.axis_index("c")
        zeros = jnp.zeros((L,), jnp.float32)
        for r in range(8):
            for c in range(nch):
                stats_v[r, pl.ds(c * L, L)] = zeros

        pltpu.sync_copy(idx_hbm.at[pl.ds(wid * pts * K, pts * K)], idx_v)
        bufs = (rows_a, rows_b)
        sems = (sem_a, sem_b)
        pltpu.async_copy(u_hbm.at[idx_v.at[pl.ds(0, idxc)]], rows_a, sem_a)

        def process(g, rows_v):
            pbase = wid * pts + g * p_sub
            pltpu.sync_copy(w_hbm.at[pl.ds(pbase, p_sub)], w_v)
            for c in range(nch):
                sl = pl.ds(c * L, L)

                def point(p, st):
                    s1, s2, cr, sw, sw2 = st
                    v0 = rows_v[p * K + 0, sl]
                    mx = v0
                    mn = v0
                    sm = v0
                    sq = v0 * v0
                    for j in range(1, K):
                        vj = rows_v[p * K + j, sl]
                        mx = jnp.maximum(mx, vj)
                        mn = jnp.minimum(mn, vj)
                        sm = sm + vj
                        sq = sq + vj * vj
                    gmax_v[p, sl] = mx
                    gmin_v[p, sl] = mn
                    wp = w_v[p, sl]
                    return (s1 + sm, s2 + sq, cr + wp * sm, sw + wp,
                            sw2 + wp * wp)

                st = lax.fori_loop(
                    0, p_sub, point, (zeros, zeros, zeros, zeros, zeros),
                    unroll=4)
                stats_v[0, sl] = stats_v[0, sl] + st[0]
                stats_v[1, sl] = stats_v[1, sl] + st[1]
                stats_v[2, sl] = stats_v[2, sl] + st[2]
                stats_v[3, sl] = stats_v[3, sl] + st[3]
                stats_v[4, sl] = stats_v[4, sl] + st[4]
            pltpu.sync_copy(gmax_v, gmax_hbm.at[pl.ds(pbase, p_sub)])
            pltpu.sync_copy(gmin_v, gmin_hbm.at[pl.ds(pbase, p_sub)])

        def pair(q, carry):
            for hb in range(2):
                g = q * 2 + hb
                pltpu.make_async_copy(
                    u_hbm.at[idx_v.at[pl.ds(0, idxc)]], bufs[hb],
                    sems[hb]).wait()

                @pl.when(g + 1 < n_sub)
                def _():
                    pltpu.async_copy(
                        u_hbm.at[idx_v.at[pl.ds((g + 1) * idxc, idxc)]],
                        bufs[1 - hb], sems[1 - hb])

                process(g, bufs[hb])
            return carry

        lax.fori_loop(0, n_sub // 2, pair, 0)
        pltpu.sync_copy(stats_v, stats_hbm.at[wid])

    return gather_kernel


def kernel(x, W, gamma, beta):
    B, C, N = x.shape
    O = W.shape[0]
    BN = B * N
    w1t = jnp.transpose(W[:, :C])
    w2t = jnp.transpose(W[:, C:])

    u, w = pl.pallas_call(
        _uw_kernel,
        grid=(B,),
        in_specs=[
            pl.BlockSpec((1, C, N), lambda b: (b, 0, 0)),
            pl.BlockSpec((C, O), lambda b: (0, 0)),
            pl.BlockSpec((C, O), lambda b: (0, 0)),
        ],
        out_specs=[
            pl.BlockSpec((1, N, O), lambda b: (b, 0, 0)),
            pl.BlockSpec((1, N, O), lambda b: (b, 0, 0)),
        ],
        out_shape=[
            jax.ShapeDtypeStruct((B, N, O), jnp.float32),
            jax.ShapeDtypeStruct((B, N, O), jnp.float32),
        ],
    )(x, w1t, w2t)

    RT = 512
    knn_call = pl.pallas_call(
        functools.partial(_knn_kernel, N, RT),
        grid=(N // RT,),
        in_specs=[
            pl.BlockSpec((C, RT), lambda r: (0, r)),
            pl.BlockSpec((C, N), lambda r: (0, 0)),
        ],
        out_specs=pl.BlockSpec((RT, K), lambda r: (r, 0)),
        out_shape=jax.ShapeDtypeStruct((N, K), jnp.int32),
        scratch_shapes=[pltpu.VMEM((1, N), jnp.float32)],
    )
    gather_call = _make_sc_gather(N, O)

    gmax_l, gmin_l, stats_l = [], [], []
    for b in range(B):
        idx_b = knn_call(x[b], x[b])
        gx, gn, st = gather_call(u[b], w[b], idx_b.reshape(N * K))
        gmax_l.append(gx)
        gmin_l.append(gn)
        stats_l.append(st)

    gmax = jnp.concatenate(gmax_l, axis=0)
    gmin = jnp.concatenate(gmin_l, axis=0)
    stats = jnp.concatenate(stats_l, axis=0)
    wf = w.reshape(BN, O)

    FT = 512
    out = pl.pallas_call(
        functools.partial(_final_kernel, N, FT, BN * K),
        grid=(BN // FT,),
        in_specs=[
            pl.BlockSpec((FT, O), lambda i: (i, 0)),
            pl.BlockSpec((FT, O), lambda i: (i, 0)),
            pl.BlockSpec((FT, O), lambda i: (i, 0)),
            pl.BlockSpec((B * NW, 8, O), lambda i: (0, 0, 0)),
            pl.BlockSpec((1, O), lambda i: (0, 0)),
            pl.BlockSpec((1, O), lambda i: (0, 0)),
        ],
        out_specs=pl.BlockSpec(
            (1, O, FT), lambda i: (i // (N // FT), 0, i % (N // FT))
        ),
        out_shape=jax.ShapeDtypeStruct((B, O, N), jnp.float32),
    )(gmax, gmin, wf, stats, gamma.reshape(1, O), beta.reshape(1, O))

    return out
